# baseline (device time: 51489 ns/iter reference)
import jax
import jax.numpy as jnp
from jax import lax
from jax.experimental import pallas as pl
from jax.experimental.pallas import tpu as pltpu

N_DEV = 8


def _nt(a, b):
    return lax.dot_general(a, b, (((1,), (1,)), ((), ())),
                           preferred_element_type=jnp.float32)


def _nn(a, b):
    return lax.dot_general(a, b, (((1,), (0,)), ((), ())),
                           preferred_element_type=jnp.float32)


def kernel(x, Wq, Wo, K_ext, V_ext):
    B, Sq, D = x.shape
    _, Skv, H, Dh = K_ext.shape
    scale = 1.0 / (Dh ** 0.5)
    BH = B * H

    def body(x_ref, wq_ref, wo_ref, k_ref, v_ref, out_ref,
             kbuf, vbuf, k_send, k_recv, v_send, v_recv):
        my = lax.axis_index("i")
        px = jnp.bitwise_xor(my, 1)
        py = jnp.bitwise_xor(my, 3)
        pz = jnp.bitwise_xor(my, 4)

        barrier_sem = pltpu.get_barrier_semaphore()
        for p in (px, py, pz):
            pl.semaphore_signal(
                barrier_sem, inc=1,
                device_id=(p,), device_id_type=pl.DeviceIdType.MESH,
            )

        k2 = k_ref[...]
        v2 = v_ref[...]
        for b in range(B):
            for h in range(H):
                kbuf[0, b, h] = k2[b * Skv:(b + 1) * Skv,
                                   h * Dh:(h + 1) * Dh].astype(jnp.bfloat16)
        pl.semaphore_wait(barrier_sem, 3)

        transfers = {
            0: (0, 1, px), 1: (0, 2, py), 2: (0, 3, pz),
            3: (2, 4, px), 4: (3, 5, py), 5: (1, 6, pz),
            6: (5, 7, px),
        }
        k_rdmas = {}
        v_rdmas = {}

        def start(t, which):
            src, dst, partner = transfers[t]
            buf, ssem, rsem, table = (
                (kbuf, k_send, k_recv, k_rdmas) if which == "k"
                else (vbuf, v_send, v_recv, v_rdmas))
            table[t] = pltpu.make_async_remote_copy(
                src_ref=buf.at[src],
                dst_ref=buf.at[dst],
                send_sem=ssem.at[t],
                recv_sem=rsem.at[t],
                device_id=(partner,),
                device_id_type=pl.DeviceIdType.MESH,
            )
            table[t].start()

        for t in (0, 1, 2):
            start(t, "k")
        for b in range(B):
            for h in range(H):
                vbuf[0, b, h] = v2[b * Skv:(b + 1) * Skv,
                                   h * Dh:(h + 1) * Dh].astype(jnp.bfloat16)
        for t in (0, 1, 2):
            start(t, "v")

        q_t = [None] * BH
        m_t = [None] * BH
        l_t = [None] * BH
        acc = [None] * BH

        def fold_chunk(slot):
            kc = kbuf[slot]
            vc = vbuf[slot]
            for i in range(BH):
                b, h = divmod(i, H)
                s = _nt(q_t[i], kc[b, h]) * scale
                if m_t[i] is None:
                    m_t[i] = jnp.max(s, axis=1, keepdims=True)
                    p = jnp.exp(s - m_t[i])
                    l_t[i] = jnp.sum(p, axis=1, keepdims=True)
                    acc[i] = _nn(p.astype(jnp.bfloat16), vc[b, h])
                else:
                    m_new = jnp.maximum(m_t[i],
                                        jnp.max(s, axis=1, keepdims=True))
                    alpha = jnp.exp(m_t[i] - m_new)
                    p = jnp.exp(s - m_new)
                    l_t[i] = l_t[i] * alpha + jnp.sum(p, axis=1,
                                                      keepdims=True)
                    acc[i] = acc[i] * alpha + _nn(p.astype(jnp.bfloat16),
                                                  vc[b, h])
                    m_t[i] = m_new

        q = jnp.dot(x_ref[...].astype(jnp.bfloat16),
                    wq_ref[...].astype(jnp.bfloat16),
                    preferred_element_type=jnp.float32)
        wo_bf = wo_ref[...].astype(jnp.bfloat16)
        for i in range(BH):
            b, h = divmod(i, H)
            q_t[i] = q[b * Sq:(b + 1) * Sq,
                       h * Dh:(h + 1) * Dh].astype(jnp.bfloat16)
        fold_chunk(0)

        k_rdmas[1].wait_recv()
        start(3, "k")
        k_rdmas[2].wait_recv()
        start(4, "k")
        k_rdmas[0].wait_recv()
        start(5, "k")
        v_rdmas[1].wait_recv()
        start(3, "v")
        v_rdmas[2].wait_recv()
        start(4, "v")
        v_rdmas[0].wait_recv()
        start(5, "v")
        fold_chunk(1)
        fold_chunk(2)
        fold_chunk(3)

        k_rdmas[4].wait_recv()
        start(6, "k")
        v_rdmas[4].wait_recv()
        start(6, "v")
        fold_chunk(5)
        k_rdmas[3].wait_recv()
        v_rdmas[3].wait_recv()
        fold_chunk(4)
        k_rdmas[5].wait_recv()
        v_rdmas[5].wait_recv()
        fold_chunk(6)
        k_rdmas[6].wait_recv()
        v_rdmas[6].wait_recv()
        fold_chunk(7)

        attn = jnp.concatenate(
            [jnp.concatenate([(acc[b * H + h] / l_t[b * H + h]
                               ).astype(jnp.bfloat16)
                              for h in range(H)], axis=1)
             for b in range(B)], axis=0)
        out_ref[...] = jnp.dot(attn, wo_bf,
                               preferred_element_type=jnp.float32)

        for t in range(7):
            k_rdmas[t].wait_send()
            v_rdmas[t].wait_send()

    out2 = pl.pallas_call(
        body,
        out_shape=jax.ShapeDtypeStruct((B * Sq, D), jnp.float32),
        in_specs=[pl.BlockSpec(memory_space=pltpu.VMEM)] * 5,
        out_specs=pl.BlockSpec(memory_space=pltpu.VMEM),
        scratch_shapes=[
            pltpu.VMEM((N_DEV, B, H, Skv, Dh), jnp.bfloat16),
            pltpu.VMEM((N_DEV, B, H, Skv, Dh), jnp.bfloat16),
            pltpu.SemaphoreType.DMA((7,)),
            pltpu.SemaphoreType.DMA((7,)),
            pltpu.SemaphoreType.DMA((7,)),
            pltpu.SemaphoreType.DMA((7,)),
        ],
        compiler_params=pltpu.CompilerParams(collective_id=0),
    )(x.reshape(B * Sq, D), Wq, Wo,
      K_ext.reshape(B * Skv, H * Dh), V_ext.reshape(B * Skv, H * Dh))
    return out2.reshape(B, Sq, D)
